# baseline (speedup 1.0000x reference)
import functools

import jax
import jax.numpy as jnp
from jax import lax
from jax.experimental import pallas as pl
from jax.experimental.pallas import tpu as pltpu
from jax.experimental.pallas import tpu_sc as plsc

_D = 128
_K = 128
_NT = 32


def _make_agg(n, ch0, ch1, k):
    npad = -(-(n + 1) // 128) * 128
    zr = npad // 16
    chm = max(ch0, ch1)
    assert ch0 % 2 == 0 and ch1 % 2 == 0 and min(ch0, ch1) >= 4

    mesh = plsc.VectorSubcoreMesh(core_axis_name="c", subcore_axis_name="s")

    @functools.partial(
        pl.kernel,
        out_type=jax.ShapeDtypeStruct((2, npad, _D), jnp.float32),
        mesh=mesh,
        scratch_types=[
            pltpu.VMEM((chm, k), jnp.int32),
            pltpu.VMEM((chm, k), jnp.int32),
            pltpu.VMEM((2, k, _D), jnp.float32),
            pltpu.VMEM_SHARED((npad, _D), jnp.float32),
        ] + [pltpu.SemaphoreType.DMA] * 4,
        compiler_params=pltpu.CompilerParams(use_tc_tiling_on_sc=False),
    )
    def agg_kernel(h_hbm, src_hbm, dst_hbm, zros_hbm, out_hbm,
                   src_v, dst_v, rows_v, acc_sh, *sems):
        sem_g, sem_s = sems[:2], sems[2:]
        c = lax.axis_index("c")
        s = lax.axis_index("s")
        w = c * 16 + s
        ch = jnp.where(c == 0, ch0, ch1)
        pltpu.sync_copy(zros_hbm, acc_sh.at[pl.ds(s * zr, zr)])
        pltpu.sync_copy(src_hbm.at[w], src_v)
        pltpu.sync_copy(dst_hbm.at[w], dst_v)
        plsc.subcore_barrier()

        def start_g(b, j):
            pltpu.async_copy(h_hbm.at[src_v.at[j]], rows_v.at[b], sem_g[b])

        def wait_g(b):
            pltpu.make_async_copy(h_hbm.at[pl.ds(0, k)], rows_v.at[b],
                                  sem_g[b]).wait()

        def start_s(b, j):
            pltpu.async_copy(rows_v.at[b], acc_sh.at[dst_v.at[j]], sem_s[b],
                             add=True)

        def wait_s(b):
            pltpu.make_async_copy(h_hbm.at[pl.ds(0, k)], rows_v.at[b],
                                  sem_s[b]).wait()

        start_g(0, 0)
        wait_g(0)
        start_g(1, 1)
        start_s(0, 0)
        wait_g(1)
        wait_s(0)
        start_g(0, 2)
        start_s(1, 1)

        def round_body(r, carry):
            j0 = 2 * r
            wait_g(0)
            wait_s(1)
            start_g(1, j0 + 1)
            start_s(0, j0)
            wait_g(1)
            wait_s(0)
            start_g(0, jnp.minimum(j0 + 2, ch - 1))
            start_s(1, j0 + 1)
            return carry

        lax.fori_loop(1, ch // 2, round_body, 0)
        wait_g(0)
        wait_s(1)
        plsc.subcore_barrier()
        pltpu.sync_copy(acc_sh.at[pl.ds(s * zr, zr)],
                        out_hbm.at[c, pl.ds(s * zr, zr)])

    return agg_kernel


def _mlp_body(h_ref, p0_ref, p1_ref, w1_ref, b1_ref, w2_ref, b2_ref,
              u_ref, s_ref, ss_ref):
    i = pl.program_id(0)
    z = h_ref[...] + p0_ref[0] + p1_ref[0]
    hid = jnp.dot(z, w1_ref[...], preferred_element_type=jnp.float32)
    hid = jnp.maximum(hid + b1_ref[0:1, :], 0.0)
    u = jnp.dot(hid, w2_ref[...], preferred_element_type=jnp.float32)
    u = u + b2_ref[0:1, :]
    u_ref[...] = u

    @pl.when(i == 0)
    def _():
        s_ref[...] = jnp.zeros_like(s_ref)
        ss_ref[...] = jnp.zeros_like(ss_ref)

    s_ref[...] += jnp.sum(u, axis=0, keepdims=True)
    ss_ref[...] += jnp.sum(u * u, axis=0, keepdims=True)


def _mlp(h, p, w1, b1t, w2, b2t, bm):
    n = h.shape[0]
    row = pl.BlockSpec((bm, _D), lambda i: (i, 0))
    rep = pl.BlockSpec((8, _D), lambda i: (0, 0))
    wsp = pl.BlockSpec((_D, _D), lambda i: (0, 0))
    ph0 = pl.BlockSpec((1, bm, _D), lambda i: (0, i, 0))
    ph1 = pl.BlockSpec((1, bm, _D), lambda i: (1, i, 0))
    return pl.pallas_call(
        _mlp_body,
        grid=(n // bm,),
        in_specs=[row, ph0, ph1, wsp, rep, wsp, rep],
        out_specs=[row, rep, rep],
        out_shape=[
            jax.ShapeDtypeStruct((n, _D), jnp.float32),
            jax.ShapeDtypeStruct((8, _D), jnp.float32),
            jax.ShapeDtypeStruct((8, _D), jnp.float32),
        ],
    )(h, p, p, w1, b1t, w2, b2t)


def _norm_body(n, u_ref, s_ref, ss_ref, g_ref, b_ref, o_ref):
    mean = s_ref[0:1, :] * (1.0 / n)
    var = ss_ref[0:1, :] * (1.0 / n) - mean * mean
    inv = lax.rsqrt(var + 1e-5)
    h = (u_ref[...] - mean) * inv * g_ref[0:1, :] + b_ref[0:1, :]
    o_ref[...] = jnp.maximum(h, 0.0)


def _norm_relu(u, s, ss, gt, bt, bm):
    n = u.shape[0]
    row = pl.BlockSpec((bm, _D), lambda i: (i, 0))
    rep = pl.BlockSpec((8, _D), lambda i: (0, 0))
    return pl.pallas_call(
        functools.partial(_norm_body, n),
        grid=(n // bm,),
        in_specs=[row, rep, rep, rep, rep],
        out_specs=row,
        out_shape=jax.ShapeDtypeStruct((n, _D), jnp.float32),
    )(u, s, ss, gt, bt)


def _final_body(n, u_ref, bt_ref, s_ref, ss_ref, g_ref, b_ref, o_ref):
    mean = s_ref[0:1, :] * (1.0 / n)
    var = ss_ref[0:1, :] * (1.0 / n) - mean * mean
    inv = lax.rsqrt(var + 1e-5)
    h = (u_ref[...] - mean) * inv * g_ref[0:1, :] + b_ref[0:1, :]
    z = h * 0.2
    zc = jnp.sum(z, axis=1, keepdims=True)
    bt = bt_ref[...]
    lane = lax.broadcasted_iota(jnp.int32, (n, _D), 1)
    mask = bt == lane
    zb = jnp.where(mask, zc, -1e30)
    segmax = jnp.max(zb, axis=0, keepdims=True)
    m = jnp.sum(jnp.where(mask, segmax, 0.0), axis=1, keepdims=True)
    e = jnp.exp(zc - m)
    seg = jnp.sum(jnp.where(mask, e, 0.0), axis=0, keepdims=True)
    den = jnp.sum(jnp.where(mask, seg, 0.0), axis=1, keepdims=True)
    o_ref[...] = e / (den + 1e-16)


def _final(u, batch2d, s, ss, gt, bt):
    n = u.shape[0]
    return pl.pallas_call(
        functools.partial(_final_body, n),
        out_shape=jax.ShapeDtypeStruct((n, 1), jnp.float32),
    )(u, batch2d, s, ss, gt, bt)


def _rep8(v):
    return jnp.broadcast_to(v.reshape(1, -1), (8, v.shape[-1]))


def kernel(x, edge_index, batch,
           W1_0, b1_0, W2_0, b2_0, gamma_0, beta_0,
           W1_1, b1_1, W2_1, b2_1, gamma_1, beta_1,
           W1_2, b1_2, W2_2, b2_2, gamma_2, beta_2):
    n, d = x.shape
    e = edge_index.shape[1]
    k = 96
    cht = -(-e // (16 * k))
    ch0 = int(round(cht * 0.375)) // 2 * 2
    ch1 = -(-(cht - ch0) // 2) * 2
    chm = max(ch0, ch1)
    e0 = 16 * ch0 * k

    def _part(v, fill):
        v0 = v[:e0].reshape(16, ch0, k)
        v0 = jnp.pad(v0, ((0, 0), (0, chm - ch0), (0, 0)), constant_values=fill)
        pad1 = 16 * ch1 * k - (e - e0)
        v1 = jnp.concatenate([v[e0:], jnp.full((pad1,), fill, jnp.int32)])
        v1 = v1.reshape(16, ch1, k)
        v1 = jnp.pad(v1, ((0, 0), (0, chm - ch1), (0, 0)), constant_values=fill)
        return jnp.concatenate([v0, v1], axis=0)

    src = _part(edge_index[0], 0)
    dst = _part(edge_index[1], n)
    npad = -(-(n + 1) // 128) * 128
    zros = jnp.zeros((npad // 16, d), jnp.float32)
    _agg = _make_agg(n, ch0, ch1, k)

    def agg(h):
        return _agg(h, src, dst, zros)

    w2_2p = jnp.pad(W2_2, ((0, 0), (0, _D - W2_2.shape[1])))
    b2_2p = jnp.pad(b2_2, (0, _D - b2_2.shape[0]))
    g2p = jnp.pad(gamma_2, (0, _D - gamma_2.shape[0]))
    be2p = jnp.pad(beta_2, (0, _D - beta_2.shape[0]))

    bm = 400
    h = x
    p = agg(h)
    u, s, ss = _mlp(h, p, W1_0, _rep8(b1_0), W2_0, _rep8(b2_0), bm)
    h = _norm_relu(u, s, ss, _rep8(gamma_0), _rep8(beta_0), bm)
    p = agg(h)
    u, s, ss = _mlp(h, p, W1_1, _rep8(b1_1), W2_1, _rep8(b2_1), bm)
    h = _norm_relu(u, s, ss, _rep8(gamma_1), _rep8(beta_1), bm)
    p = agg(h)
    u, s, ss = _mlp(h, p, W1_2, _rep8(b1_2), w2_2p, _rep8(b2_2p), bm)
    return _final(u, batch.reshape(n, 1), s, ss, _rep8(g2p), _rep8(be2p))


# speedup vs baseline: 1.2634x; 1.2634x over previous
import functools

import jax
import jax.numpy as jnp
from jax import lax
from jax.experimental import pallas as pl
from jax.experimental.pallas import tpu as pltpu
from jax.experimental.pallas import tpu_sc as plsc

_D = 128
_K = 128
_NT = 32


def _make_agg(n, ch0, ch1, k):
    npad = -(-(n + 1) // 128) * 128
    zr = npad // 16
    chm = max(ch0, ch1)
    assert ch0 % 2 == 0 and ch1 % 2 == 0 and min(ch0, ch1) >= 4

    mesh = plsc.VectorSubcoreMesh(core_axis_name="c", subcore_axis_name="s")

    @functools.partial(
        pl.kernel,
        out_type=jax.ShapeDtypeStruct((2, npad, _D), jnp.float32),
        mesh=mesh,
        scratch_types=[
            pltpu.VMEM((chm, k), jnp.int32),
            pltpu.VMEM((chm, k), jnp.int32),
            pltpu.VMEM((2, k, _D), jnp.float32),
            pltpu.VMEM_SHARED((npad, _D), jnp.float32),
        ] + [pltpu.SemaphoreType.DMA] * 4,
        compiler_params=pltpu.CompilerParams(use_tc_tiling_on_sc=False),
    )
    def agg_kernel(h_hbm, src_hbm, dst_hbm, zros_hbm, out_hbm,
                   src_v, dst_v, rows_v, acc_sh, *sems):
        sem_g, sem_s = sems[:2], sems[2:]
        c = lax.axis_index("c")
        s = lax.axis_index("s")
        w = c * 16 + s
        ch = jnp.where(c == 0, ch0, ch1)
        pltpu.sync_copy(zros_hbm, acc_sh.at[pl.ds(s * zr, zr)])
        pltpu.sync_copy(src_hbm.at[w], src_v)
        pltpu.sync_copy(dst_hbm.at[w], dst_v)
        plsc.subcore_barrier()

        def start_g(b, j):
            pltpu.async_copy(h_hbm.at[src_v.at[j]], rows_v.at[b], sem_g[b])

        def wait_g(b):
            pltpu.make_async_copy(h_hbm.at[pl.ds(0, k)], rows_v.at[b],
                                  sem_g[b]).wait()

        def start_s(b, j):
            pltpu.async_copy(rows_v.at[b], acc_sh.at[dst_v.at[j]], sem_s[b],
                             add=True)

        def wait_s(b):
            pltpu.make_async_copy(h_hbm.at[pl.ds(0, k)], rows_v.at[b],
                                  sem_s[b]).wait()

        start_g(0, 0)
        wait_g(0)
        start_g(1, 1)
        start_s(0, 0)
        wait_g(1)
        wait_s(0)
        start_g(0, 2)
        start_s(1, 1)

        def round_body(r, carry):
            j0 = 2 * r
            wait_g(0)
            wait_s(1)
            start_g(1, j0 + 1)
            start_s(0, j0)
            wait_g(1)
            wait_s(0)
            start_g(0, jnp.minimum(j0 + 2, ch - 1))
            start_s(1, j0 + 1)
            return carry

        lax.fori_loop(1, ch // 2, round_body, 0)
        wait_g(0)
        wait_s(1)
        plsc.subcore_barrier()
        pltpu.sync_copy(acc_sh.at[pl.ds(s * zr, zr)],
                        out_hbm.at[c, pl.ds(s * zr, zr)])

    return agg_kernel


def _mlp_body(h_ref, p0_ref, p1_ref, w1_ref, b1_ref, w2_ref, b2_ref,
              u_ref, s_ref, ss_ref):
    i = pl.program_id(0)
    z = h_ref[...] + p0_ref[0] + p1_ref[0]
    hid = jnp.dot(z, w1_ref[...], preferred_element_type=jnp.float32)
    hid = jnp.maximum(hid + b1_ref[0:1, :], 0.0)
    u = jnp.dot(hid, w2_ref[...], preferred_element_type=jnp.float32)
    u = u + b2_ref[0:1, :]
    u_ref[...] = u

    @pl.when(i == 0)
    def _():
        s_ref[...] = jnp.zeros_like(s_ref)
        ss_ref[...] = jnp.zeros_like(ss_ref)

    s_ref[...] += jnp.sum(u, axis=0, keepdims=True)
    ss_ref[...] += jnp.sum(u * u, axis=0, keepdims=True)


def _mlp(h, p, w1, b1t, w2, b2t, bm):
    n = h.shape[0]
    row = pl.BlockSpec((bm, _D), lambda i: (i, 0))
    rep = pl.BlockSpec((8, _D), lambda i: (0, 0))
    wsp = pl.BlockSpec((_D, _D), lambda i: (0, 0))
    ph0 = pl.BlockSpec((1, bm, _D), lambda i: (0, i, 0))
    ph1 = pl.BlockSpec((1, bm, _D), lambda i: (1, i, 0))
    return pl.pallas_call(
        _mlp_body,
        grid=(n // bm,),
        in_specs=[row, ph0, ph1, wsp, rep, wsp, rep],
        out_specs=[row, rep, rep],
        out_shape=[
            jax.ShapeDtypeStruct((n, _D), jnp.float32),
            jax.ShapeDtypeStruct((8, _D), jnp.float32),
            jax.ShapeDtypeStruct((8, _D), jnp.float32),
        ],
    )(h, p, p, w1, b1t, w2, b2t)


def _norm_body(n, u_ref, s_ref, ss_ref, g_ref, b_ref, o_ref):
    mean = s_ref[0:1, :] * (1.0 / n)
    var = ss_ref[0:1, :] * (1.0 / n) - mean * mean
    inv = lax.rsqrt(var + 1e-5)
    h = (u_ref[...] - mean) * inv * g_ref[0:1, :] + b_ref[0:1, :]
    o_ref[...] = jnp.maximum(h, 0.0)


def _norm_relu(u, s, ss, gt, bt, bm):
    n = u.shape[0]
    row = pl.BlockSpec((bm, _D), lambda i: (i, 0))
    rep = pl.BlockSpec((8, _D), lambda i: (0, 0))
    return pl.pallas_call(
        functools.partial(_norm_body, n),
        grid=(n // bm,),
        in_specs=[row, rep, rep, rep, rep],
        out_specs=row,
        out_shape=jax.ShapeDtypeStruct((n, _D), jnp.float32),
    )(u, s, ss, gt, bt)


def _final_body(n, u_ref, bt_ref, s_ref, ss_ref, g_ref, b_ref, o_ref):
    mean = s_ref[0:1, :] * (1.0 / n)
    var = ss_ref[0:1, :] * (1.0 / n) - mean * mean
    inv = lax.rsqrt(var + 1e-5)
    h = (u_ref[...] - mean) * inv * g_ref[0:1, :] + b_ref[0:1, :]
    z = h * 0.2
    zc = jnp.sum(z, axis=1, keepdims=True)
    bt = bt_ref[...]
    lane = lax.broadcasted_iota(jnp.int32, (n, _D), 1)
    mask = bt == lane
    zb = jnp.where(mask, zc, -1e30)
    segmax = jnp.max(zb, axis=0, keepdims=True)
    m = jnp.sum(jnp.where(mask, segmax, 0.0), axis=1, keepdims=True)
    e = jnp.exp(zc - m)
    seg = jnp.sum(jnp.where(mask, e, 0.0), axis=0, keepdims=True)
    den = jnp.sum(jnp.where(mask, seg, 0.0), axis=1, keepdims=True)
    o_ref[...] = e / (den + 1e-16)


def _final(u, batch2d, s, ss, gt, bt):
    n = u.shape[0]
    return pl.pallas_call(
        functools.partial(_final_body, n),
        out_shape=jax.ShapeDtypeStruct((n, 1), jnp.float32),
    )(u, batch2d, s, ss, gt, bt)


def _rep8(v):
    return jnp.broadcast_to(v.reshape(1, -1), (8, v.shape[-1]))


def kernel(x, edge_index, batch,
           W1_0, b1_0, W2_0, b2_0, gamma_0, beta_0,
           W1_1, b1_1, W2_1, b2_1, gamma_1, beta_1,
           W1_2, b1_2, W2_2, b2_2, gamma_2, beta_2):
    n, d = x.shape
    e = edge_index.shape[1]
    k = 96
    cht = -(-e // (16 * k))
    ch0 = int(round(cht * 0.625)) // 2 * 2
    ch1 = -(-(cht - ch0) // 2) * 2
    chm = max(ch0, ch1)
    e0 = 16 * ch0 * k

    def _part(v, fill):
        v0 = v[:e0].reshape(16, ch0, k)
        v0 = jnp.pad(v0, ((0, 0), (0, chm - ch0), (0, 0)), constant_values=fill)
        pad1 = 16 * ch1 * k - (e - e0)
        v1 = jnp.concatenate([v[e0:], jnp.full((pad1,), fill, jnp.int32)])
        v1 = v1.reshape(16, ch1, k)
        v1 = jnp.pad(v1, ((0, 0), (0, chm - ch1), (0, 0)), constant_values=fill)
        return jnp.concatenate([v0, v1], axis=0)

    src = _part(edge_index[0], 0)
    dst = _part(edge_index[1], n)
    npad = -(-(n + 1) // 128) * 128
    zros = jnp.zeros((npad // 16, d), jnp.float32)
    _agg = _make_agg(n, ch0, ch1, k)

    def agg(h):
        return _agg(h, src, dst, zros)

    w2_2p = jnp.pad(W2_2, ((0, 0), (0, _D - W2_2.shape[1])))
    b2_2p = jnp.pad(b2_2, (0, _D - b2_2.shape[0]))
    g2p = jnp.pad(gamma_2, (0, _D - gamma_2.shape[0]))
    be2p = jnp.pad(beta_2, (0, _D - beta_2.shape[0]))

    bm = 400
    h = x
    p = agg(h)
    u, s, ss = _mlp(h, p, W1_0, _rep8(b1_0), W2_0, _rep8(b2_0), bm)
    h = _norm_relu(u, s, ss, _rep8(gamma_0), _rep8(beta_0), bm)
    p = agg(h)
    u, s, ss = _mlp(h, p, W1_1, _rep8(b1_1), W2_1, _rep8(b2_1), bm)
    h = _norm_relu(u, s, ss, _rep8(gamma_1), _rep8(beta_1), bm)
    p = agg(h)
    u, s, ss = _mlp(h, p, W1_2, _rep8(b1_2), w2_2p, _rep8(b2_2p), bm)
    return _final(u, batch.reshape(n, 1), s, ss, _rep8(g2p), _rep8(be2p))
